# Initial kernel scaffold; baseline (speedup 1.0000x reference)
import jax
import jax.numpy as jnp
from jax.experimental import pallas as pl


def kernel(x, ms_mask, Wq, Wkv, Wproj, bproj, norm1_s, norm1_b, norm2_s, norm2_b, ln1_s, ln1_b, Wg, We, be):
    raise NotImplementedError("write your pallas kernel here")


# speedup vs baseline: 1.8394x; 1.8394x over previous
import jax
import jax.numpy as jnp
from jax.experimental import pallas as pl
from jax.experimental.pallas import tpu as pltpu

M, B, S, D, E, K, H = 3, 4, 128, 512, 8, 2, 8
N = M * S
HD = D // H
NEG = -1e30


def _rowsum(x):
    n = x.shape[-1]
    acc = x[:, 0:128]
    for i in range(1, n // 128):
        acc = acc + x[:, 128 * i:128 * (i + 1)]
    p = acc[:, 0:8]
    for i in range(1, 16):
        p = p + acc[:, 8 * i:8 * (i + 1)]
    p = p[:, 0:4] + p[:, 4:8]
    p = p[:, 0:2] + p[:, 2:4]
    return p[:, 0:1] + p[:, 1:2]


def _ln(x, scale, bias, eps=1e-5):
    mu = _rowsum(x) * (1.0 / x.shape[-1])
    d = x - mu
    var = _rowsum(d * d) * (1.0 / x.shape[-1])
    return d / jnp.sqrt(var + eps) * scale + bias


def _gelu(x):
    return 0.5 * x * (1.0 + jax.lax.erf(x * jnp.float32(0.7071067811865476)))


def _fused(x_ref, mask_ref, Wq_ref, Wkv_ref, Wproj_ref, bproj_ref,
           n1s_ref, n1b_ref, n2s_ref, n2b_ref, l1s_ref, l1b_ref,
           Wg_ref, We_ref, be_ref, out_ref, conf_ref):
    xb = x_ref[0]
    h = _ln(xb, n1s_ref[...], n1b_ref[...])
    q = jnp.dot(h, Wq_ref[...], preferred_element_type=jnp.float32)
    kv = jnp.dot(h, Wkv_ref[...], preferred_element_type=jnp.float32)
    scale = jnp.float32(HD ** -0.5)
    o_heads = []
    for hh in range(H):
        qh = q[:, hh * HD:(hh + 1) * HD] * scale
        kh = kv[:, hh * HD:(hh + 1) * HD]
        vh = kv[:, D + hh * HD:D + (hh + 1) * HD]
        s = jax.lax.dot_general(qh, kh, (((1,), (1,)), ((), ())),
                                preferred_element_type=jnp.float32)
        s = s - jnp.max(s, axis=-1, keepdims=True)
        p = jnp.exp(s)
        p = p / _rowsum(p)
        o_heads.append(jnp.dot(p, vh, preferred_element_type=jnp.float32))
    o = jnp.concatenate(o_heads, axis=1)
    o = jnp.dot(o, Wproj_ref[...], preferred_element_type=jnp.float32) + bproj_ref[...]
    xres = o + o
    hi = _ln(xres, n2s_ref[...], n2b_ref[...])
    gl = jnp.dot(hi, Wg_ref[...], preferred_element_type=jnp.float32)
    ids = jax.lax.broadcasted_iota(jnp.int32, (N, E), 1)
    v1 = jnp.max(gl, axis=-1, keepdims=True)
    i1 = jnp.min(jnp.where(gl == v1, ids, E), axis=-1, keepdims=True)
    gl2 = jnp.where(ids == i1, NEG, gl)
    v2 = jnp.max(gl2, axis=-1, keepdims=True)
    i2 = jnp.min(jnp.where(gl2 == v2, ids, E), axis=-1, keepdims=True)
    e2 = jnp.exp(v2 - v1)
    denom = 1.0 + e2
    c1 = 1.0 / denom
    c2 = e2 / denom
    comb = jnp.zeros((N, D), jnp.float32)
    for e in range(E):
        ye = _gelu(jnp.dot(hi, We_ref[e], preferred_element_type=jnp.float32)
                   + be_ref[e, :][None, :])
        ge = c1 * (i1 == e).astype(jnp.float32) + c2 * (i2 == e).astype(jnp.float32)
        comb = comb + ge * ye
    comb = comb * (1.0 - mask_ref[0])
    out_ref[0] = _ln(xres + comb, l1s_ref[...], l1b_ref[...])
    conf_ref[0] = jnp.concatenate([c1, c2], axis=1)


def kernel(x, ms_mask, Wq, Wkv, Wproj, bproj, norm1_s, norm1_b, norm2_s,
           norm2_b, ln1_s, ln1_b, Wg, We, be):
    xcat = x.transpose(1, 0, 2, 3).reshape(B, N, D)
    maskN = jnp.repeat(ms_mask.astype(jnp.float32).T, S, axis=1)[..., None]
    row = lambda v: v.reshape(1, D)
    full = lambda a: pl.BlockSpec(a.shape, lambda b: (0,) * a.ndim)
    out, conf = pl.pallas_call(
        _fused,
        grid=(B,),
        in_specs=[
            pl.BlockSpec((1, N, D), lambda b: (b, 0, 0)),
            pl.BlockSpec((1, N, 1), lambda b: (b, 0, 0)),
        ] + [full(a) for a in (Wq, Wkv, Wproj, row(bproj), row(norm1_s),
                               row(norm1_b), row(norm2_s), row(norm2_b),
                               row(ln1_s), row(ln1_b), Wg, We, be)],
        out_specs=[
            pl.BlockSpec((1, N, D), lambda b: (b, 0, 0)),
            pl.BlockSpec((1, N, K), lambda b: (b, 0, 0)),
        ],
        out_shape=[
            jax.ShapeDtypeStruct((B, N, D), jnp.float32),
            jax.ShapeDtypeStruct((B, N, K), jnp.float32),
        ],
    )(xcat, maskN, Wq, Wkv, Wproj, row(bproj), row(norm1_s), row(norm1_b),
      row(norm2_s), row(norm2_b), row(ln1_s), row(ln1_b), Wg, We, be)
    confs = conf.reshape(B, M, S, K).transpose(1, 0, 2, 3)
    return out, confs


# speedup vs baseline: 3.1567x; 1.7162x over previous
import jax
import jax.numpy as jnp
from jax.experimental import pallas as pl
from jax.experimental.pallas import tpu as pltpu

M, B, S, D, E, K, H = 3, 4, 128, 512, 8, 2, 8
N = M * S
HD = D // H
NEG = -1e30


def _rowsum(x):
    n = x.shape[-1]
    acc = x[:, 0:128]
    for i in range(1, n // 128):
        acc = acc + x[:, 128 * i:128 * (i + 1)]
    t = acc.T
    p = t[0:8]
    for i in range(1, 16):
        p = p + t[8 * i:8 * (i + 1)]
    p = p[0:4] + p[4:8]
    p = p[0:2] + p[2:4]
    p = p[0:1] + p[1:2]
    return p.T


def _ln(x, scale, bias, eps=1e-5):
    mu = _rowsum(x) * (1.0 / x.shape[-1])
    d = x - mu
    var = _rowsum(d * d) * (1.0 / x.shape[-1])
    return d / jnp.sqrt(var + eps) * scale + bias


def _ln_fast(x, scale, bias, eps=1e-5):
    mu = jnp.mean(x, axis=-1, keepdims=True)
    d = x - mu
    var = jnp.mean(d * d, axis=-1, keepdims=True)
    return d / jnp.sqrt(var + eps) * scale + bias


def _gelu(x):
    return 0.5 * x * (1.0 + jax.lax.erf(x * jnp.float32(0.7071067811865476)))


def _fused(x_ref, mask_ref, Wq_ref, Wkv_ref, Wproj_ref, bproj_ref,
           n1s_ref, n1b_ref, n2s_ref, n2b_ref, l1s_ref, l1b_ref,
           Wg_ref, We_ref, be_ref, out_ref, conf_ref):
    xb = x_ref[0]
    h = _ln(xb, n1s_ref[...], n1b_ref[...])
    q = jnp.dot(h, Wq_ref[...], preferred_element_type=jnp.float32)
    kv = jnp.dot(h, Wkv_ref[...], preferred_element_type=jnp.float32)
    scale = jnp.float32(HD ** -0.5)
    o_heads = []
    for hh in range(H):
        qh = q[:, hh * HD:(hh + 1) * HD] * scale
        kh = kv[:, hh * HD:(hh + 1) * HD]
        vh = kv[:, D + hh * HD:D + (hh + 1) * HD]
        s = jax.lax.dot_general(qh, kh, (((1,), (1,)), ((), ())),
                                preferred_element_type=jnp.float32)
        s = s - jnp.max(s, axis=-1, keepdims=True)
        p = jnp.exp(s)
        p = p / _rowsum(p)
        o_heads.append(jnp.dot(p, vh, preferred_element_type=jnp.float32))
    o = jnp.concatenate(o_heads, axis=1)
    o = jnp.dot(o, Wproj_ref[...], preferred_element_type=jnp.float32) + bproj_ref[...]
    xres = o + o
    hi = _ln(xres, n2s_ref[...], n2b_ref[...])
    gl = jnp.dot(hi, Wg_ref[...], preferred_element_type=jnp.float32)
    ids = jax.lax.broadcasted_iota(jnp.int32, (N, E), 1)
    v1 = jnp.max(gl, axis=-1, keepdims=True)
    i1 = jnp.min(jnp.where(gl == v1, ids, E), axis=-1, keepdims=True)
    gl2 = jnp.where(ids == i1, NEG, gl)
    v2 = jnp.max(gl2, axis=-1, keepdims=True)
    i2 = jnp.min(jnp.where(gl2 == v2, ids, E), axis=-1, keepdims=True)
    e2 = jnp.exp(v2 - v1)
    denom = 1.0 + e2
    c1 = 1.0 / denom
    c2 = e2 / denom
    comb = jnp.zeros((N, D), jnp.float32)
    for e in range(E):
        ye = _gelu(jnp.dot(hi, We_ref[e], preferred_element_type=jnp.float32)
                   + be_ref[e, :][None, :])
        ge = c1 * (i1 == e).astype(jnp.float32) + c2 * (i2 == e).astype(jnp.float32)
        comb = comb + ge * ye
    comb = comb * (1.0 - mask_ref[0])
    out_ref[0] = _ln_fast(xres + comb, l1s_ref[...], l1b_ref[...])
    conf_ref[0] = jnp.concatenate([c1, c2], axis=1)


def kernel(x, ms_mask, Wq, Wkv, Wproj, bproj, norm1_s, norm1_b, norm2_s,
           norm2_b, ln1_s, ln1_b, Wg, We, be):
    xcat = x.transpose(1, 0, 2, 3).reshape(B, N, D)
    maskN = jnp.repeat(ms_mask.astype(jnp.float32).T, S, axis=1)[..., None]
    row = lambda v: v.reshape(1, D)
    full = lambda a: pl.BlockSpec(a.shape, lambda b: (0,) * a.ndim)
    out, conf = pl.pallas_call(
        _fused,
        grid=(B,),
        in_specs=[
            pl.BlockSpec((1, N, D), lambda b: (b, 0, 0)),
            pl.BlockSpec((1, N, 1), lambda b: (b, 0, 0)),
        ] + [full(a) for a in (Wq, Wkv, Wproj, row(bproj), row(norm1_s),
                               row(norm1_b), row(norm2_s), row(norm2_b),
                               row(ln1_s), row(ln1_b), Wg, We, be)],
        out_specs=[
            pl.BlockSpec((1, N, D), lambda b: (b, 0, 0)),
            pl.BlockSpec((1, N, K), lambda b: (b, 0, 0)),
        ],
        out_shape=[
            jax.ShapeDtypeStruct((B, N, D), jnp.float32),
            jax.ShapeDtypeStruct((B, N, K), jnp.float32),
        ],
    )(xcat, maskN, Wq, Wkv, Wproj, row(bproj), row(norm1_s), row(norm1_b),
      row(norm2_s), row(norm2_b), row(ln1_s), row(ln1_b), Wg, We, be)
    confs = conf.reshape(B, M, S, K).transpose(1, 0, 2, 3)
    return out, confs


# speedup vs baseline: 3.3398x; 1.0580x over previous
import jax
import jax.numpy as jnp
from jax.experimental import pallas as pl
from jax.experimental.pallas import tpu as pltpu

M, B, S, D, E, K, H = 3, 4, 128, 512, 8, 2, 8
N = M * S
HD = D // H
NEG = -1e30


def _rowsum(x):
    n = x.shape[-1]
    acc = x[:, 0:128]
    for i in range(1, n // 128):
        acc = acc + x[:, 128 * i:128 * (i + 1)]
    t = acc.T
    p = t[0:8]
    for i in range(1, 16):
        p = p + t[8 * i:8 * (i + 1)]
    p = p[0:4] + p[4:8]
    p = p[0:2] + p[2:4]
    p = p[0:1] + p[1:2]
    return p.T


def _ln(x, scale, bias, eps=1e-5):
    mu = _rowsum(x) * (1.0 / x.shape[-1])
    d = x - mu
    var = _rowsum(d * d) * (1.0 / x.shape[-1])
    return d / jnp.sqrt(var + eps) * scale + bias


def _ln_fast(x, scale, bias, eps=1e-5):
    mu = jnp.mean(x, axis=-1, keepdims=True)
    d = x - mu
    var = jnp.mean(d * d, axis=-1, keepdims=True)
    return d / jnp.sqrt(var + eps) * scale + bias


def _gelu(x):
    return 0.5 * x * (1.0 + jax.lax.erf(x * jnp.float32(0.7071067811865476)))


def _fused(x_ref, mask_ref, Wq_ref, Wkv_ref, Wproj_ref, bproj_ref,
           n1s_ref, n1b_ref, n2s_ref, n2b_ref, l1s_ref, l1b_ref,
           Wg_ref, We_ref, be_ref, out_ref, conf_ref):
    xb = x_ref[:, 0].reshape(N, D)
    h = _ln(xb, n1s_ref[...], n1b_ref[...])
    q = jnp.dot(h, Wq_ref[...], preferred_element_type=jnp.float32)
    kv = jnp.dot(h, Wkv_ref[...], preferred_element_type=jnp.float32)
    scale = jnp.float32(HD ** -0.5)
    o_heads = []
    for hh in range(H):
        qh = q[:, hh * HD:(hh + 1) * HD] * scale
        kh = kv[:, hh * HD:(hh + 1) * HD]
        vh = kv[:, D + hh * HD:D + (hh + 1) * HD]
        s = jax.lax.dot_general(qh, kh, (((1,), (1,)), ((), ())),
                                preferred_element_type=jnp.float32)
        s = s - jnp.max(s, axis=-1, keepdims=True)
        p = jnp.exp(s)
        p = p / _rowsum(p)
        o_heads.append(jnp.dot(p, vh, preferred_element_type=jnp.float32))
    o = jnp.concatenate(o_heads, axis=1)
    o = jnp.dot(o, Wproj_ref[...], preferred_element_type=jnp.float32) + bproj_ref[...]
    xres = o + o
    hi = _ln(xres, n2s_ref[...], n2b_ref[...])
    gl = jnp.dot(hi, Wg_ref[...], preferred_element_type=jnp.float32)
    ids = jax.lax.broadcasted_iota(jnp.int32, (N, E), 1)
    v1 = jnp.max(gl, axis=-1, keepdims=True)
    i1 = jnp.min(jnp.where(gl == v1, ids, E), axis=-1, keepdims=True)
    gl2 = jnp.where(ids == i1, NEG, gl)
    v2 = jnp.max(gl2, axis=-1, keepdims=True)
    i2 = jnp.min(jnp.where(gl2 == v2, ids, E), axis=-1, keepdims=True)
    e2 = jnp.exp(v2 - v1)
    denom = 1.0 + e2
    c1 = 1.0 / denom
    c2 = e2 / denom
    comb = jnp.zeros((N, D), jnp.float32)
    for e in range(E):
        ye = _gelu(jnp.dot(hi, We_ref[e], preferred_element_type=jnp.float32)
                   + be_ref[e, :][None, :])
        ge = c1 * (i1 == e).astype(jnp.float32) + c2 * (i2 == e).astype(jnp.float32)
        comb = comb + ge * ye
    comb = comb * (1.0 - mask_ref[0])
    out_ref[0] = _ln_fast(xres + comb, l1s_ref[...], l1b_ref[...])
    conf_ref[0] = jnp.concatenate([c1, c2], axis=1)


def kernel(x, ms_mask, Wq, Wkv, Wproj, bproj, norm1_s, norm1_b, norm2_s,
           norm2_b, ln1_s, ln1_b, Wg, We, be):
    maskN = jnp.repeat(ms_mask.astype(jnp.float32).T, S, axis=1)[..., None]
    row = lambda v: v.reshape(1, D)
    full = lambda a: pl.BlockSpec(a.shape, lambda b: (0,) * a.ndim)
    out, conf = pl.pallas_call(
        _fused,
        grid=(B,),
        in_specs=[
            pl.BlockSpec((M, 1, S, D), lambda b: (0, b, 0, 0)),
            pl.BlockSpec((1, N, 1), lambda b: (b, 0, 0)),
        ] + [full(a) for a in (Wq, Wkv, Wproj, row(bproj), row(norm1_s),
                               row(norm1_b), row(norm2_s), row(norm2_b),
                               row(ln1_s), row(ln1_b), Wg, We, be)],
        out_specs=[
            pl.BlockSpec((1, N, D), lambda b: (b, 0, 0)),
            pl.BlockSpec((1, N, K), lambda b: (b, 0, 0)),
        ],
        out_shape=[
            jax.ShapeDtypeStruct((B, N, D), jnp.float32),
            jax.ShapeDtypeStruct((B, N, K), jnp.float32),
        ],
    )(x, maskN, Wq, Wkv, Wproj, row(bproj), row(norm1_s), row(norm1_b),
      row(norm2_s), row(norm2_b), row(ln1_s), row(ln1_b), Wg, We, be)
    confs = conf.reshape(B, M, S, K).transpose(1, 0, 2, 3)
    return out, confs
